# baseline (speedup 1.0000x reference)
import jax
import jax.numpy as jnp
from jax import lax
from jax.experimental import pallas as pl
from jax.experimental.pallas import tpu as pltpu
from jax.experimental.pallas import tpu_sc as plsc

_NC = 2
_NS = 16
_NW = _NC * _NS
_L = 16
_CR = 16
_NBUF = 4
_LOOKAHEAD = 2


def _sc_body(x_hbm, pos_hbm, alpha_hbm, pe_hbm, out_hbm, comb0_v, comb1_v,
             tmp_v, xa_v, xb_v, xc_v, xd_v, alpha_v, sl0, sl1, sl2, sl3,
             ss0, ss1, ss2, ss3, st0, st1):
    b, s, d = x_hbm.shape
    rows_per_w = s // _NW
    n_chunks = rows_per_w // _CR
    vecs = (_CR * d) // _L

    wid = lax.axis_index("s") * _NC + lax.axis_index("c")
    base = wid * rows_per_w

    pltpu.sync_copy(alpha_hbm, alpha_v)
    a_vec = alpha_v[...]
    a = 1.0 / (1.0 + jnp.exp(-a_vec))
    om_a = 1.0 - a

    combs = (comb0_v, comb1_v)
    tsems = (st0, st1)
    xbufs = (xa_v, xb_v, xc_v, xd_v)
    lsems = (sl0, sl1, sl2, sl3)
    ssems = (ss0, ss1, ss2, ss3)
    n_items = n_chunks * b

    def item_cb(k):
        return k // b, k % b

    def start_load(k):
        c, bi = item_cb(k)
        p = k % _NBUF
        return pltpu.async_copy(
            x_hbm.at[bi, pl.ds(base + c * _CR, _CR), :], xbufs[p], lsems[p])

    def start_store(k):
        c, bi = item_cb(k)
        p = k % _NBUF
        return pltpu.async_copy(
            xbufs[p], out_hbm.at[bi, pl.ds(base + c * _CR, _CR), :], ssems[p])

    def start_tables(c):
        r0 = base + c * _CR
        q = c % 2
        return (
            pltpu.async_copy(pos_hbm.at[pl.ds(r0, _CR), :], combs[q],
                             tsems[q]),
            pltpu.async_copy(pe_hbm.at[pl.ds(r0, _CR), :], tmp_v, tsems[q]),
        )

    loads = {}
    stores = {}
    for k in range(min(_LOOKAHEAD, n_items)):
        loads[k] = start_load(k)
    tloads = {0: start_tables(0)}

    for k in range(n_items):
        c, bi = item_cb(k)
        comb_v = combs[c % 2]
        if bi == 0:
            tloads[c][0].wait()
            tloads[c][1].wait()

            @plsc.parallel_loop(0, vecs, unroll=8)
            def _(i):
                r = i >> 6
                cc = pl.multiple_of((i & 63) << 4, _L)
                comb_v[r, pl.ds(cc, _L)] = (
                    a * comb_v[r, pl.ds(cc, _L)]
                    + om_a * tmp_v[r, pl.ds(cc, _L)])

        if bi == 1 and c + 1 < n_chunks:
            tloads[c + 1] = start_tables(c + 1)

        nxt = k + _LOOKAHEAD
        if nxt < n_items:
            if nxt - _NBUF >= 0:
                stores[nxt - _NBUF].wait()
            loads[nxt] = start_load(nxt)
        loads[k].wait()

        xv = xbufs[k % _NBUF]

        @plsc.parallel_loop(0, vecs, unroll=16)
        def _(i):
            r = i >> 6
            cc = pl.multiple_of((i & 63) << 4, _L)
            plsc.addupdate(xv.at[r, pl.ds(cc, _L)], comb_v[r, pl.ds(cc, _L)])

        stores[k] = start_store(k)

    for k in range(max(0, n_items - _NBUF), n_items):
        stores[k].wait()


def kernel(x, pos_emb, alpha, pe):
    b, s, d = x.shape
    alpha1 = jnp.full((_L,), alpha, dtype=jnp.float32)
    mesh = plsc.VectorSubcoreMesh(core_axis_name="c", subcore_axis_name="s")
    f = pl.kernel(
        _sc_body,
        out_type=jax.ShapeDtypeStruct((b, s, d), jnp.float32),
        mesh=mesh,
        scratch_types=(
            [pltpu.VMEM((_CR, d), jnp.float32)] * (3 + _NBUF)
            + [pltpu.VMEM((_L,), jnp.float32)]
            + [pltpu.SemaphoreType.DMA] * (2 * _NBUF + 2)
        ),
    )
    return f(x, pos_emb[:s], alpha1, pe[:s])


# speedup vs baseline: 1.0301x; 1.0301x over previous
import jax
import jax.numpy as jnp
from jax import lax
from jax.experimental import pallas as pl
from jax.experimental.pallas import tpu as pltpu
from jax.experimental.pallas import tpu_sc as plsc

_NC = 2
_NS = 16
_NW = _NC * _NS
_L = 16
_CR = 16
_NBUF = 4
_LOOKAHEAD = 2


def _sc_body(x_hbm, pos_hbm, alpha_hbm, pe_hbm, out_hbm, comb0_v, comb1_v,
             tmp_v, xa_v, xb_v, xc_v, xd_v, alpha_v, sl0, sl1, sl2, sl3,
             ss0, ss1, ss2, ss3, st0, st1):
    b, s, d = x_hbm.shape
    rows_per_w = s // _NW
    n_chunks = rows_per_w // _CR
    vecs = (_CR * d) // _L

    wid = lax.axis_index("s") * _NC + lax.axis_index("c")
    base = wid * rows_per_w

    pltpu.sync_copy(alpha_hbm, alpha_v)
    a_vec = alpha_v[...]
    a = 1.0 / (1.0 + jnp.exp(-a_vec))
    om_a = 1.0 - a

    combs = (comb0_v, comb1_v)
    tsems = (st0, st1)
    xbufs = (xa_v, xb_v, xc_v, xd_v)
    lsems = (sl0, sl1, sl2, sl3)
    ssems = (ss0, ss1, ss2, ss3)
    n_items = n_chunks * b

    def item_cb(k):
        return k // b, k % b

    def start_load(k):
        c, bi = item_cb(k)
        p = k % _NBUF
        return pltpu.async_copy(
            x_hbm.at[bi, pl.ds(base + c * _CR, _CR), :], xbufs[p], lsems[p])

    def start_store(k):
        c, bi = item_cb(k)
        p = k % _NBUF
        return pltpu.async_copy(
            xbufs[p], out_hbm.at[bi, pl.ds(base + c * _CR, _CR), :], ssems[p])

    def start_tables(c):
        r0 = base + c * _CR
        q = c % 2
        return (
            pltpu.async_copy(pos_hbm.at[pl.ds(r0, _CR), :], combs[q],
                             tsems[q]),
            pltpu.async_copy(pe_hbm.at[pl.ds(r0, _CR), :], tmp_v, tsems[q]),
        )

    loads = {}
    stores = {}
    for k in range(min(_LOOKAHEAD, n_items)):
        loads[k] = start_load(k)
    tloads = {0: start_tables(0)}

    for k in range(n_items):
        c, bi = item_cb(k)
        comb_v = combs[c % 2]
        if bi == 0:
            tloads[c][0].wait()
            tloads[c][1].wait()

            @plsc.parallel_loop(0, vecs, unroll=8)
            def _(i):
                r = i >> 6
                cc = pl.multiple_of((i & 63) << 4, _L)
                comb_v[r, pl.ds(cc, _L)] = (
                    a * comb_v[r, pl.ds(cc, _L)]
                    + om_a * tmp_v[r, pl.ds(cc, _L)])

            if c + 1 < n_chunks:
                tloads[c + 1] = start_tables(c + 1)


        nxt = k + _LOOKAHEAD
        if nxt < n_items:
            if nxt - _NBUF >= 0:
                stores[nxt - _NBUF].wait()
            loads[nxt] = start_load(nxt)
        loads[k].wait()

        xv = xbufs[k % _NBUF]

        @plsc.parallel_loop(0, vecs, unroll=8)
        def _(i):
            r = i >> 6
            cc = pl.multiple_of((i & 63) << 4, _L)
            plsc.addupdate(xv.at[r, pl.ds(cc, _L)], comb_v[r, pl.ds(cc, _L)])

        stores[k] = start_store(k)

    for k in range(max(0, n_items - _NBUF), n_items):
        stores[k].wait()


def kernel(x, pos_emb, alpha, pe):
    b, s, d = x.shape
    alpha1 = jnp.full((_L,), alpha, dtype=jnp.float32)
    mesh = plsc.VectorSubcoreMesh(core_axis_name="c", subcore_axis_name="s")
    f = pl.kernel(
        _sc_body,
        out_type=jax.ShapeDtypeStruct((b, s, d), jnp.float32),
        mesh=mesh,
        scratch_types=(
            [pltpu.VMEM((_CR, d), jnp.float32)] * (3 + _NBUF)
            + [pltpu.VMEM((_L,), jnp.float32)]
            + [pltpu.SemaphoreType.DMA] * (2 * _NBUF + 2)
        ),
    )
    return f(x, pos_emb[:s], alpha1, pe[:s])
